# baseline (speedup 1.0000x reference)
import functools

import jax
import jax.numpy as jnp
from jax.experimental import pallas as pl

_N = 10000
_E = 320000
_NUM_GRAPHS = 64


def _ee_matmul(eat, We):
    M, K = eat.shape
    D = We.shape[1]
    BM = 1000
    assert M % BM == 0

    def body(a_ref, w_ref, o_ref):
        o_ref[...] = jax.lax.dot_general(
            a_ref[...], w_ref[...], (((1,), (0,)), ((), ())),
            preferred_element_type=jnp.float32)

    return pl.pallas_call(
        body,
        grid=(M // BM,),
        in_specs=[
            pl.BlockSpec((BM, K), lambda i: (i, 0)),
            pl.BlockSpec((K, D), lambda i: (0, 0)),
        ],
        out_specs=pl.BlockSpec((BM, D), lambda i: (i, 0)),
        out_shape=jax.ShapeDtypeStruct((M, D), jnp.float32),
    )(eat, We)


def _gat_layer(h, edge_index, ea, p, li):
    n = h.shape[0]
    src0, dst0 = edge_index[0], edge_index[1]
    deg = jax.ops.segment_sum(jnp.ones((ea.shape[0],), jnp.float32), dst0,
                              num_segments=n)
    loop_attr = jax.ops.segment_sum(ea, dst0, num_segments=n) / jnp.maximum(
        deg, 1.0)[:, None]
    ar = jnp.arange(n)
    src = jnp.concatenate([src0, ar])
    dst = jnp.concatenate([dst0, ar])
    eat = jnp.concatenate([ea, loop_attr], axis=0)
    xl = h @ p['Wl%d' % li] + p['bl%d' % li]
    xr = h @ p['Wr%d' % li] + p['br%d' % li]
    ee = _ee_matmul(eat, p['We%d' % li])
    m = jax.nn.leaky_relu(xl[src] + xr[dst] + ee, negative_slope=0.2)
    alpha = m @ p['att%d' % li]
    amax = jax.ops.segment_max(alpha, dst, num_segments=n)
    amax = jnp.where(jnp.isfinite(amax), amax, 0.0)
    ex = jnp.exp(alpha - amax[dst])
    denom = jax.ops.segment_sum(ex, dst, num_segments=n)
    w = ex / jnp.maximum(denom[dst], 1e-16)
    out = jax.ops.segment_sum(xl[src] * w[:, None], dst, num_segments=n)
    return out + p['bias%d' % li]


def kernel(x, edge_index, edge_attr, batch, atom_table, bond_table, bool_table,
           Wl0, bl0, Wr0, br0, We0, att0, bias0,
           Wl1, bl1, Wr1, br1, We1, att1, bias1,
           Wl2, bl2, Wr2, br2, We2, att2, bias2,
           Wl3, bl3, Wr3, br3, We3, att3, bias3):
    p = {
        'Wl0': Wl0, 'bl0': bl0, 'Wr0': Wr0, 'br0': br0, 'We0': We0, 'att0': att0, 'bias0': bias0,
        'Wl1': Wl1, 'bl1': bl1, 'Wr1': Wr1, 'br1': br1, 'We1': We1, 'att1': att1, 'bias1': bias1,
        'Wl2': Wl2, 'bl2': bl2, 'Wr2': Wr2, 'br2': br2, 'We2': We2, 'att2': att2, 'bias2': bias2,
        'Wl3': Wl3, 'bl3': bl3, 'Wr3': Wr3, 'br3': br3, 'We3': We3, 'att3': att3, 'bias3': bias3,
    }
    ai = x[:, 0].astype(jnp.int32)
    ab = x[:, 9].astype(jnp.int32)
    bt = edge_attr[:, 0].astype(jnp.int32)
    bc = edge_attr[:, 2].astype(jnp.int32)
    ba = edge_attr[:, 3].astype(jnp.int32)
    h = jnp.concatenate([atom_table[ai], x[:, 1:9], bool_table[ab]], axis=1)
    ea = jnp.concatenate([bond_table[bt], edge_attr[:, 1:2], bool_table[bc],
                          bool_table[ba]], axis=1)
    for li in range(4):
        h = _gat_layer(h, edge_index, ea, p, li)
        h = jax.nn.relu(h)
    s = jax.ops.segment_sum(h, batch, num_segments=_NUM_GRAPHS)
    cnt = jax.ops.segment_sum(jnp.ones((h.shape[0],), jnp.float32), batch,
                              num_segments=_NUM_GRAPHS)
    return s / jnp.maximum(cnt, 1.0)[:, None]
